# baseline (speedup 1.0000x reference)
import functools

import jax
import jax.numpy as jnp
from jax import lax
from jax.experimental import pallas as pl
from jax.experimental.pallas import tpu as pltpu
from jax.experimental.pallas import tpu_sc as plsc

H = 768
E = 8
K = 2
FF = 4 * H
T = 2048
BLK = 128
CAP = T * K + E * BLK
NB = CAP // BLK

_NC = 2
_NS = 16
A_PER = (T * K) // _NS
Z_PER = CAP // _NS
SL_PER = CAP // (_NC * _NS)
GCH = SL_PER // 2
TOK_PER = T // (_NC * _NS)
CTOK = 16


def _gelu_exact(h):
    return 0.5 * h * (1.0 + lax.erf(h * 0.7071067811865476))


def _gate_body(x_ref, wg_ref, bg_ref, counts_ref, ends_ref, dests_ref, wts_ref):
    xf = x_ref[...]
    logits = jnp.dot(xf, wg_ref[...], preferred_element_type=jnp.float32)
    logits = logits + bg_ref[...]
    eidx = lax.broadcasted_iota(jnp.int32, (T, E), 1)
    m1 = jnp.max(logits, axis=1, keepdims=True)
    i1 = jnp.min(jnp.where(logits == m1, eidx, E), axis=1, keepdims=True)
    oh1 = (eidx == i1).astype(jnp.float32)
    l2 = jnp.where(eidx == i1, -1e30, logits)
    m2 = jnp.max(l2, axis=1, keepdims=True)
    i2 = jnp.min(jnp.where(l2 == m2, eidx, E), axis=1, keepdims=True)
    oh2 = (eidx == i2).astype(jnp.float32)
    w1 = 1.0 / (1.0 + jnp.exp(m2 - m1))
    w2 = 1.0 - w1
    oh = oh1 + oh2
    counts = jnp.sum(oh, axis=0, keepdims=True)
    counts_ref[...] = counts
    pc = jnp.floor((counts + (BLK - 1)) * (1.0 / BLK)) * BLK
    ui = (lax.broadcasted_iota(jnp.int32, (E, E), 0)
          <= lax.broadcasted_iota(jnp.int32, (E, E), 1)).astype(jnp.float32)
    ends = jnp.dot(pc, ui, preferred_element_type=jnp.float32)
    pe = ends - pc
    ends_ref[...] = ends.astype(jnp.int32)
    tr = lax.broadcasted_iota(jnp.int32, (T, T), 0)
    tc = lax.broadcasted_iota(jnp.int32, (T, T), 1)
    ltri = (tc < tr).astype(jnp.bfloat16)
    excl = jnp.dot(ltri, oh.astype(jnp.bfloat16),
                   preferred_element_type=jnp.float32)
    rank1 = jnp.sum(excl * oh1, axis=1, keepdims=True)
    rank2 = jnp.sum(excl * oh2, axis=1, keepdims=True)
    d1 = jnp.sum(pe * oh1, axis=1, keepdims=True) + rank1
    d2 = jnp.sum(pe * oh2, axis=1, keepdims=True) + rank2
    dests_ref[...] = jnp.concatenate([d1, d2], axis=1).astype(jnp.int32)
    wts_ref[...] = jnp.concatenate([w1, w2], axis=1)


def _gate_call(flat, Wg, bg2):
    return pl.pallas_call(
        _gate_body,
        out_shape=(
            jax.ShapeDtypeStruct((1, E), jnp.float32),
            jax.ShapeDtypeStruct((1, E), jnp.int32),
            jax.ShapeDtypeStruct((T, K), jnp.int32),
            jax.ShapeDtypeStruct((T, K), jnp.float32),
        ),
    )(flat, Wg, bg2)


def _dispatch_body(dests_hbm, wts_hbm, x_hbm, xs_hbm, sw_hbm,
                   didx, tokv, wv, zi, zf, swv, gidx, rows, stok_sp, sw_sp, sem):
    c = lax.axis_index("c")
    s = lax.axis_index("s")
    for j in range(Z_PER // 16):
        zi[pl.ds(j * 16, 16)] = jnp.zeros((16,), jnp.int32)
        zf[pl.ds(j * 16, 16)] = jnp.zeros((16,), jnp.float32)
    pltpu.sync_copy(zi, stok_sp.at[pl.ds(s * Z_PER, Z_PER)])
    pltpu.sync_copy(zf, sw_sp.at[pl.ds(s * Z_PER, Z_PER)])
    plsc.subcore_barrier()
    abase = s * A_PER
    pltpu.sync_copy(dests_hbm.at[pl.ds(abase, A_PER)], didx)
    pltpu.sync_copy(wts_hbm.at[pl.ds(abase, A_PER)], wv)
    for j in range(A_PER // 16):
        a0 = lax.iota(jnp.int32, 16) + (abase + j * 16)
        tokv[pl.ds(j * 16, 16)] = jnp.right_shift(a0, 1)
    pltpu.sync_copy(tokv, stok_sp.at[didx], add=True)
    pltpu.sync_copy(wv, sw_sp.at[didx], add=True)
    plsc.subcore_barrier()
    slot_base = (c * _NS + s) * SL_PER
    pltpu.sync_copy(sw_sp.at[pl.ds(slot_base, SL_PER)], swv)
    pltpu.sync_copy(swv, sw_hbm.at[pl.ds(slot_base, SL_PER)])
    for g in range(SL_PER // GCH):
        o = slot_base + g * GCH
        pltpu.sync_copy(stok_sp.at[pl.ds(o, GCH)], gidx)
        pltpu.async_copy(x_hbm.at[gidx], rows, sem).wait()
        pltpu.sync_copy(rows, xs_hbm.at[pl.ds(o, GCH)])


def _expert_idx(b, ends_ref):
    acc = jnp.int32(0)
    for j in range(E):
        acc = acc + (ends_ref[j] <= b * BLK).astype(jnp.int32)
    return jnp.minimum(acc, E - 1)


def _ffn_body(ends_ref, xs_ref, w1_ref, b1_ref, w2_ref, b2_ref, sw_ref, eo_ref):
    xb = xs_ref[...].astype(jnp.bfloat16)
    h = jnp.dot(xb, w1_ref[0], preferred_element_type=jnp.float32) + b1_ref[0]
    h = _gelu_exact(h).astype(jnp.bfloat16)
    o = jnp.dot(h, w2_ref[0], preferred_element_type=jnp.float32) + b2_ref[0]
    eo_ref[...] = o * sw_ref[...]


def _ffn_call(ends8, xs, W1, b1, W2, b2, sw2):
    grid_spec = pltpu.PrefetchScalarGridSpec(
        num_scalar_prefetch=1,
        grid=(NB,),
        in_specs=[
            pl.BlockSpec((BLK, H), lambda b, ends: (b, 0)),
            pl.BlockSpec((1, H, FF), lambda b, ends: (_expert_idx(b, ends), 0, 0)),
            pl.BlockSpec((1, 1, FF), lambda b, ends: (_expert_idx(b, ends), 0, 0)),
            pl.BlockSpec((1, FF, H), lambda b, ends: (_expert_idx(b, ends), 0, 0)),
            pl.BlockSpec((1, 1, H), lambda b, ends: (_expert_idx(b, ends), 0, 0)),
            pl.BlockSpec((BLK, 1), lambda b, ends: (b, 0)),
        ],
        out_specs=pl.BlockSpec((BLK, H), lambda b, ends: (b, 0)),
    )
    return pl.pallas_call(
        _ffn_body,
        grid_spec=grid_spec,
        out_shape=jax.ShapeDtypeStruct((CAP, H), jnp.float32),
    )(ends8, xs, W1, b1, W2, b2, sw2)


def _combine_body(dests_hbm, x_hbm, eo_hbm, out_hbm, gidx, grows, xb, ob, sem):
    c = lax.axis_index("c")
    s = lax.axis_index("s")
    tbase = (c * _NS + s) * TOK_PER

    def chunk(ci, carry):
        base = tbase + ci * CTOK
        pltpu.sync_copy(dests_hbm.at[pl.ds(2 * base, 2 * CTOK)], gidx)
        pltpu.async_copy(eo_hbm.at[gidx], grows, sem).wait()
        pltpu.sync_copy(x_hbm.at[pl.ds(base, CTOK)], xb)
        for i in range(CTOK):
            for cc in range(H // 16):
                sl = pl.ds(cc * 16, 16)
                ob[i, sl] = xb[i, sl] + grows[2 * i, sl] + grows[2 * i + 1, sl]
        pltpu.sync_copy(ob, out_hbm.at[pl.ds(base, CTOK)])
        return carry

    lax.fori_loop(0, TOK_PER // CTOK, chunk, 0)


@functools.lru_cache(maxsize=1)
def _sc_kernels():
    mesh = plsc.VectorSubcoreMesh(core_axis_name="c", subcore_axis_name="s")
    dispatch = pl.kernel(
        _dispatch_body,
        out_type=(
            jax.ShapeDtypeStruct((CAP, H), jnp.float32),
            jax.ShapeDtypeStruct((CAP,), jnp.float32),
        ),
        mesh=mesh,
        scratch_types=[
            pltpu.VMEM((A_PER,), jnp.int32),
            pltpu.VMEM((A_PER,), jnp.int32),
            pltpu.VMEM((A_PER,), jnp.float32),
            pltpu.VMEM((Z_PER,), jnp.int32),
            pltpu.VMEM((Z_PER,), jnp.float32),
            pltpu.VMEM((SL_PER,), jnp.float32),
            pltpu.VMEM((GCH,), jnp.int32),
            pltpu.VMEM((GCH, H), jnp.float32),
            pltpu.VMEM_SHARED((CAP,), jnp.int32),
            pltpu.VMEM_SHARED((CAP,), jnp.float32),
            pltpu.SemaphoreType.DMA,
        ],
    )
    combine = pl.kernel(
        _combine_body,
        out_type=jax.ShapeDtypeStruct((T, H), jnp.float32),
        mesh=mesh,
        scratch_types=[
            pltpu.VMEM((2 * CTOK,), jnp.int32),
            pltpu.VMEM((2 * CTOK, H), jnp.float32),
            pltpu.VMEM((CTOK, H), jnp.float32),
            pltpu.VMEM((CTOK, H), jnp.float32),
            pltpu.SemaphoreType.DMA,
        ],
    )
    return dispatch, combine


def kernel(x, Wg, bg, W1, b1, W2, b2):
    B, S, Hd = x.shape
    flat = x.reshape(T, H)
    counts, ends, dests, wts = _gate_call(flat, Wg, bg.reshape(1, E))
    dests_flat = dests.reshape(T * K)
    wts_flat = wts.reshape(T * K)
    dispatch, combine = _sc_kernels()
    xs, sw = dispatch(dests_flat, wts_flat, flat)
    eo = _ffn_call(ends.reshape(E), xs, W1.astype(jnp.bfloat16),
                   b1.reshape(E, 1, FF), W2.astype(jnp.bfloat16),
                   b2.reshape(E, 1, H), sw.reshape(CAP, 1))
    out = combine(dests_flat, flat, eo)
    return out.reshape(B, S, Hd), counts.reshape(E)


# speedup vs baseline: 1.3353x; 1.3353x over previous
import functools

import jax
import jax.numpy as jnp
from jax import lax
from jax.experimental import pallas as pl
from jax.experimental.pallas import tpu as pltpu
from jax.experimental.pallas import tpu_sc as plsc

H = 768
E = 8
K = 2
FF = 4 * H
T = 2048
BLK = 128
CAP = T * K + E * BLK
NB = CAP // BLK

_NC = 2
_NS = 16
A_PER = (T * K) // _NS
Z_PER = CAP // _NS
SL_PER = CAP // (_NC * _NS)
GCH = SL_PER // 2
TOK_PER = T // (_NC * _NS)
CTOK = 16


def _gelu_exact(h):
    return 0.5 * h * (1.0 + lax.erf(h * 0.7071067811865476))


def _gate_body(x_ref, wg_ref, bg_ref, counts_ref, ends_ref, dests_ref, wts_ref):
    xf = x_ref[...]
    logits = jnp.dot(xf, wg_ref[...], preferred_element_type=jnp.float32)
    logits = logits + bg_ref[...]
    eidx = lax.broadcasted_iota(jnp.int32, (T, E), 1)
    m1 = jnp.max(logits, axis=1, keepdims=True)
    i1 = jnp.min(jnp.where(logits == m1, eidx, E), axis=1, keepdims=True)
    oh1 = (eidx == i1).astype(jnp.float32)
    l2 = jnp.where(eidx == i1, -1e30, logits)
    m2 = jnp.max(l2, axis=1, keepdims=True)
    i2 = jnp.min(jnp.where(l2 == m2, eidx, E), axis=1, keepdims=True)
    oh2 = (eidx == i2).astype(jnp.float32)
    w1 = 1.0 / (1.0 + jnp.exp(m2 - m1))
    w2 = 1.0 - w1
    oh = oh1 + oh2
    counts = jnp.sum(oh, axis=0, keepdims=True)
    counts_ref[...] = counts
    pc = jnp.floor((counts + (BLK - 1)) * (1.0 / BLK)) * BLK
    ui = (lax.broadcasted_iota(jnp.int32, (E, E), 0)
          <= lax.broadcasted_iota(jnp.int32, (E, E), 1)).astype(jnp.float32)
    ends = jnp.dot(pc, ui, preferred_element_type=jnp.float32)
    pe = ends - pc
    ends_ref[...] = ends.astype(jnp.int32)
    tr = lax.broadcasted_iota(jnp.int32, (T, T), 0)
    tc = lax.broadcasted_iota(jnp.int32, (T, T), 1)
    ltri = (tc < tr).astype(jnp.bfloat16)
    excl = jnp.dot(ltri, oh.astype(jnp.bfloat16),
                   preferred_element_type=jnp.float32)
    rank1 = jnp.sum(excl * oh1, axis=1, keepdims=True)
    rank2 = jnp.sum(excl * oh2, axis=1, keepdims=True)
    d1 = jnp.sum(pe * oh1, axis=1, keepdims=True) + rank1
    d2 = jnp.sum(pe * oh2, axis=1, keepdims=True) + rank2
    dests_ref[...] = jnp.concatenate([d1, d2], axis=1).astype(jnp.int32)
    wts_ref[...] = jnp.concatenate([w1, w2], axis=1)


def _gate_call(flat, Wg, bg2):
    return pl.pallas_call(
        _gate_body,
        out_shape=(
            jax.ShapeDtypeStruct((1, E), jnp.float32),
            jax.ShapeDtypeStruct((1, E), jnp.int32),
            jax.ShapeDtypeStruct((T, K), jnp.int32),
            jax.ShapeDtypeStruct((T, K), jnp.float32),
        ),
    )(flat, Wg, bg2)


def _dispatch_body(dests_hbm, wts_hbm, x_hbm, xs_hbm, sw_hbm,
                   didx, tokv, wv, zi, zf, swv, gidx0, gidx1, rows0, rows1,
                   stok_sp, sw_sp, sem0, sem1, sem2, sem3):
    c = lax.axis_index("c")
    s = lax.axis_index("s")
    abase = s * A_PER
    h_d = pltpu.async_copy(dests_hbm.at[pl.ds(abase, A_PER)], didx, sem2)
    h_w = pltpu.async_copy(wts_hbm.at[pl.ds(abase, A_PER)], wv, sem3)
    for j in range(Z_PER // 16):
        zi[pl.ds(j * 16, 16)] = jnp.zeros((16,), jnp.int32)
        zf[pl.ds(j * 16, 16)] = jnp.zeros((16,), jnp.float32)
    h_z0 = pltpu.async_copy(zi, stok_sp.at[pl.ds(s * Z_PER, Z_PER)], sem0)
    h_z1 = pltpu.async_copy(zf, sw_sp.at[pl.ds(s * Z_PER, Z_PER)], sem1)
    for j in range(A_PER // 16):
        a0 = lax.iota(jnp.int32, 16) + (abase + j * 16)
        tokv[pl.ds(j * 16, 16)] = jnp.right_shift(a0, 1)
    h_z0.wait()
    h_z1.wait()
    plsc.subcore_barrier()
    h_d.wait()
    h_w.wait()
    h_s0 = pltpu.async_copy(tokv, stok_sp.at[didx], sem0, add=True)
    h_s1 = pltpu.async_copy(wv, sw_sp.at[didx], sem1, add=True)
    h_s0.wait()
    h_s1.wait()
    plsc.subcore_barrier()
    slot_base = (c * _NS + s) * SL_PER
    o0 = slot_base
    o1 = slot_base + GCH
    pltpu.sync_copy(stok_sp.at[pl.ds(o0, GCH)], gidx0)
    h_g0 = pltpu.async_copy(x_hbm.at[gidx0], rows0, sem0)
    pltpu.sync_copy(stok_sp.at[pl.ds(o1, GCH)], gidx1)
    h_g1 = pltpu.async_copy(x_hbm.at[gidx1], rows1, sem1)
    pltpu.sync_copy(sw_sp.at[pl.ds(slot_base, SL_PER)], swv)
    h_sw = pltpu.async_copy(swv, sw_hbm.at[pl.ds(slot_base, SL_PER)], sem2)
    h_g0.wait()
    h_x0 = pltpu.async_copy(rows0, xs_hbm.at[pl.ds(o0, GCH)], sem3)
    h_g1.wait()
    h_x1 = pltpu.async_copy(rows1, xs_hbm.at[pl.ds(o1, GCH)], sem0)
    h_sw.wait()
    h_x0.wait()
    h_x1.wait()


def _expert_idx(b, ends_ref):
    acc = jnp.int32(0)
    for j in range(E):
        acc = acc + (ends_ref[j] <= b * BLK).astype(jnp.int32)
    return jnp.minimum(acc, E - 1)


def _ffn_body(ends_ref, xs_ref, w1_ref, b1_ref, w2_ref, b2_ref, sw_ref, eo_ref):
    xb = xs_ref[...]
    h = jnp.dot(xb, w1_ref[0], preferred_element_type=jnp.float32) + b1_ref[0]
    h = _gelu_exact(h)
    o = jnp.dot(h, w2_ref[0], preferred_element_type=jnp.float32) + b2_ref[0]
    eo_ref[...] = o * sw_ref[...]


def _ffn_call(ends8, xs, W1, b1, W2, b2, sw2):
    grid_spec = pltpu.PrefetchScalarGridSpec(
        num_scalar_prefetch=1,
        grid=(NB,),
        in_specs=[
            pl.BlockSpec((BLK, H), lambda b, ends: (b, 0)),
            pl.BlockSpec((1, H, FF), lambda b, ends: (_expert_idx(b, ends), 0, 0)),
            pl.BlockSpec((1, 1, FF), lambda b, ends: (_expert_idx(b, ends), 0, 0)),
            pl.BlockSpec((1, FF, H), lambda b, ends: (_expert_idx(b, ends), 0, 0)),
            pl.BlockSpec((1, 1, H), lambda b, ends: (_expert_idx(b, ends), 0, 0)),
            pl.BlockSpec((BLK, 1), lambda b, ends: (b, 0)),
        ],
        out_specs=pl.BlockSpec((BLK, H), lambda b, ends: (b, 0)),
    )
    return pl.pallas_call(
        _ffn_body,
        grid_spec=grid_spec,
        out_shape=jax.ShapeDtypeStruct((CAP, H), jnp.float32),
    )(ends8, xs, W1, b1, W2, b2, sw2)


_NCH = TOK_PER // CTOK


def _combine_body(d0_hbm, d1_hbm, x_hbm, eo_hbm, out_hbm,
                  d0v, d1v, g0b, g1b, xbb, obb, *sems):
    c = lax.axis_index("c")
    s = lax.axis_index("s")
    tbase = (c * _NS + s) * TOK_PER
    pltpu.sync_copy(d0_hbm.at[pl.ds(tbase, TOK_PER)], d0v)
    pltpu.sync_copy(d1_hbm.at[pl.ds(tbase, TOK_PER)], d1v)

    def fire(g):
        k = g % 2
        sl = pl.ds(g * CTOK, CTOK)
        return (
            pltpu.async_copy(eo_hbm.at[d0v.at[sl]], g0b.at[k], sems[4 * k + 0]),
            pltpu.async_copy(eo_hbm.at[d1v.at[sl]], g1b.at[k], sems[4 * k + 1]),
            pltpu.async_copy(x_hbm.at[pl.ds(tbase + g * CTOK, CTOK)],
                             xbb.at[k], sems[4 * k + 2]),
        )

    inflight = {0: fire(0)}
    writes = {}
    for g in range(_NCH):
        k = g % 2
        if g + 1 < _NCH:
            inflight[g + 1] = fire(g + 1)
        for h in inflight.pop(g):
            h.wait()

        def tok_body(i, carry, k=k):
            for cc in range(H // 16):
                sl = pl.ds(cc * 16, 16)
                obb[k, i, sl] = xbb[k, i, sl] + g0b[k, i, sl] + g1b[k, i, sl]
            return carry

        lax.fori_loop(0, CTOK, tok_body, 0)
        if g >= 2:
            writes.pop(g - 2).wait()
        writes[g] = pltpu.async_copy(
            obb.at[k], out_hbm.at[pl.ds(tbase + g * CTOK, CTOK)],
            sems[4 * k + 3])
    for h in writes.values():
        h.wait()


@functools.lru_cache(maxsize=1)
def _sc_kernels():
    mesh = plsc.VectorSubcoreMesh(core_axis_name="c", subcore_axis_name="s")
    dispatch = pl.kernel(
        _dispatch_body,
        out_type=(
            jax.ShapeDtypeStruct((CAP, H), jnp.float32),
            jax.ShapeDtypeStruct((CAP,), jnp.float32),
        ),
        mesh=mesh,
        scratch_types=[
            pltpu.VMEM((A_PER,), jnp.int32),
            pltpu.VMEM((A_PER,), jnp.int32),
            pltpu.VMEM((A_PER,), jnp.float32),
            pltpu.VMEM((Z_PER,), jnp.int32),
            pltpu.VMEM((Z_PER,), jnp.float32),
            pltpu.VMEM((SL_PER,), jnp.float32),
            pltpu.VMEM((GCH,), jnp.int32),
            pltpu.VMEM((GCH,), jnp.int32),
            pltpu.VMEM((GCH, H), jnp.float32),
            pltpu.VMEM((GCH, H), jnp.float32),
            pltpu.VMEM_SHARED((CAP,), jnp.int32),
            pltpu.VMEM_SHARED((CAP,), jnp.float32),
            pltpu.SemaphoreType.DMA,
            pltpu.SemaphoreType.DMA,
            pltpu.SemaphoreType.DMA,
            pltpu.SemaphoreType.DMA,
        ],
    )
    combine = pl.kernel(
        _combine_body,
        out_type=jax.ShapeDtypeStruct((T, H), jnp.float32),
        mesh=mesh,
        scratch_types=[
            pltpu.VMEM((TOK_PER,), jnp.int32),
            pltpu.VMEM((TOK_PER,), jnp.int32),
            pltpu.VMEM((2, CTOK, H), jnp.float32),
            pltpu.VMEM((2, CTOK, H), jnp.float32),
            pltpu.VMEM((2, CTOK, H), jnp.float32),
            pltpu.VMEM((2, CTOK, H), jnp.float32),
        ] + [pltpu.SemaphoreType.DMA] * 8,
    )
    return dispatch, combine


def kernel(x, Wg, bg, W1, b1, W2, b2):
    B, S, Hd = x.shape
    flat = x.reshape(T, H)
    counts, ends, dests, wts = _gate_call(flat, Wg, bg.reshape(1, E))
    dests_flat = dests.reshape(T * K)
    wts_flat = wts.reshape(T * K)
    dispatch, combine = _sc_kernels()
    xs, sw = dispatch(dests_flat, wts_flat, flat)
    eo = _ffn_call(ends.reshape(E), xs, W1, b1.reshape(E, 1, FF),
                   W2, b2.reshape(E, 1, H), sw.reshape(CAP, 1))
    out = combine(dests[:, 0], dests[:, 1], flat, eo)
    return out.reshape(B, S, Hd), counts.reshape(E)
